# baseline (speedup 1.0000x reference)
import functools
import math

import jax
import jax.numpy as jnp
from jax import lax
from jax.experimental import pallas as pl
from jax.experimental.pallas import tpu as pltpu
from jax.experimental.pallas import tpu_sc as plsc

N = 10000
E = 320000
C = 128
ALPHA = 0.1
BETA = float(math.log(0.5 / 1 + 1))

NC = 2
NS = 16

K = 80
EPT = E // (NC * NS)
NCHUNK = EPT // K
EPC = E // NS
NCHUNK_D = EPC // K
NB = 3
NI = 8

RPT = 624
RLAST = 640
EBLK = 80


def _deg_to_d(v):
  v = v + 1.0
  s = v
  for _ in range(16):
    s = 0.5 * (s + v / s)
  return 1.0 / s


def _make_fused_kernel():
  mesh = plsc.VectorSubcoreMesh(core_axis_name="c", subcore_axis_name="s")

  @functools.partial(
      pl.kernel,
      mesh=mesh,
      out_type=(
          jax.ShapeDtypeStruct((NC, N, C), jnp.float32),
          jax.ShapeDtypeStruct((N, C), jnp.float32),
      ),
      scratch_types=[
          pltpu.VMEM_SHARED((N,), jnp.float32),
          pltpu.VMEM_SHARED((N, C), jnp.float32),
          pltpu.VMEM((10, K), jnp.int32),
          pltpu.VMEM((10, K), jnp.int32),
          pltpu.VMEM((NB, K, C), jnp.float32),
          pltpu.VMEM((EBLK, C), jnp.float32),
          pltpu.VMEM((EBLK,), jnp.float32),
          pltpu.VMEM((K,), jnp.float32),
          pltpu.VMEM((640,), jnp.float32),
          pltpu.SemaphoreType.DMA((21,)),
          pltpu.SemaphoreType.DMA((NB,)),
          pltpu.SemaphoreType.DMA((NB,)),
      ],
      compiler_params=pltpu.CompilerParams(needs_layout_passes=False),
  )
  def fused(x_hbm, src_hbm, dst_hbm, hp_hbm, xs_hbm,
            cnt_shared, h_shared, sidx, didx, rows, tbuf, dbuf, ones, zsrc,
            isem, gsem, ssem):
    c = lax.axis_index("c")
    s = lax.axis_index("s")
    w = c * NS + s

    def _fill_z(i, _):
      zsrc[pl.ds(i * 16, 16)] = jnp.zeros((16,), jnp.float32)
      return _
    lax.fori_loop(0, 640 // 16, _fill_z, None)
    for j in range(K // 16):
      ones[pl.ds(j * 16, 16)] = jnp.ones((16,), jnp.float32)

    @pl.when(s < NS - 1)
    def _():
      pltpu.sync_copy(zsrc.at[pl.ds(0, 640)], cnt_shared.at[pl.ds(s * 640, 640)])

    @pl.when(s == NS - 1)
    def _():
      pltpu.sync_copy(zsrc.at[pl.ds(0, 400)], cnt_shared.at[pl.ds(9600, 400)])

    plsc.subcore_barrier()

    GSZ = 10
    NGRP = NCHUNK_D // GSZ

    def _dslot(j):
      return sidx.at[j] if j < GSZ else didx.at[j - GSZ]

    def _deg_load_desc(i, sl):
      return pltpu.make_async_copy(
          dst_hbm.at[pl.ds(s * EPC + i * K, K)], _dslot(sl), isem.at[sl])

    def _deg_scat_desc(sl):
      return pltpu.make_async_copy(ones, cnt_shared.at[_dslot(sl)],
                                   isem.at[2 * GSZ])

    for j in range(GSZ):
      _deg_load_desc(j, j).start()

    def _deg_halfgroup(g, base):
      for j in range(GSZ):
        _deg_load_desc(g * GSZ + j, base + j).wait()
      for j in range(GSZ):
        _deg_scat_desc(base + j).start(add=True)
      for j in range(GSZ):
        _deg_scat_desc(base + j).wait()

    def _deg_group(g2, _):
      ga = 2 * g2
      for j in range(GSZ):
        _deg_load_desc((ga + 1) * GSZ + j, GSZ + j).start()
      _deg_halfgroup(ga, 0)
      for j in range(GSZ):
        _deg_load_desc((ga + 2) * GSZ + j, j).start()
      _deg_halfgroup(ga + 1, GSZ)
      return _

    lax.fori_loop(0, (NGRP - 1) // 2, _deg_group, None)
    _deg_halfgroup(NGRP - 1, 0)
    plsc.subcore_barrier()

    def _load_d(r0, nb):
      pltpu.sync_copy(cnt_shared.at[pl.ds(r0, nb)], dbuf.at[pl.ds(0, nb)])
      def _dchunk(k, _):
        sl = pl.ds(k * 16, 16)
        dbuf[sl] = _deg_to_d(dbuf[sl])
        return _
      lax.fori_loop(0, nb // 16, _dchunk, None)

    def _scale_rows(nb):
      def _row(r, _):
        idx = jnp.broadcast_to(r, (16,)).astype(jnp.int32)
        dv = plsc.load_gather(dbuf, [idx])
        for j in range(C // 16):
          sl = pl.ds(j * 16, 16)
          tbuf[r, sl] = tbuf[r, sl] * dv
        return _
      lax.fori_loop(0, nb, _row, None)

    def _prescale(r0, nb):
      _load_d(r0, nb)
      pltpu.sync_copy(x_hbm.at[pl.ds(r0, nb), :], tbuf.at[pl.ds(0, nb), :])
      _scale_rows(nb)
      pltpu.sync_copy(tbuf.at[pl.ds(0, nb), :], xs_hbm.at[pl.ds(r0, nb), :])

      @pl.when(c == 0)
      def _():
        pltpu.sync_copy(tbuf.at[pl.ds(0, nb), :],
                        h_shared.at[pl.ds(r0, nb), :])

    def _zero_fill(r0, nb):
      pltpu.sync_copy(tbuf.at[pl.ds(0, nb), :], h_shared.at[pl.ds(r0, nb), :])

    def _over_blocks(fn):
      @pl.when(s < NS - 1)
      def _():
        def _b(bk, _):
          fn(s * RPT + bk * EBLK, EBLK)
          return _
        lax.fori_loop(0, RPT // EBLK, _b, None)
        fn(s * RPT + (RPT // EBLK) * EBLK, RPT - (RPT // EBLK) * EBLK)

      @pl.when(s == NS - 1)
      def _():
        def _b(bk, _):
          fn((NS - 1) * RPT + bk * EBLK, EBLK)
          return _
        lax.fori_loop(0, RLAST // EBLK, _b, None)

    _over_blocks(_prescale)

    @pl.when(c == 1)
    def _():
      def _zt(r, _):
        for j in range(C // 16):
          tbuf[r, pl.ds(j * 16, 16)] = jnp.zeros((16,), jnp.float32)
        return _
      lax.fori_loop(0, EBLK, _zt, None)
      _over_blocks(_zero_fill)

    plsc.subcore_barrier()

    def _idx_descs(i, j):
      d1 = pltpu.make_async_copy(src_hbm.at[pl.ds(w * EPT + i * K, K)],
                                 sidx.at[j], isem.at[j])
      d2 = pltpu.make_async_copy(dst_hbm.at[pl.ds(w * EPT + i * K, K)],
                                 didx.at[j], isem.at[j])
      return (d1, d2)

    def _gather_desc(j):
      return pltpu.make_async_copy(xs_hbm.at[sidx.at[j]], rows.at[j % NB],
                                   gsem.at[j % NB])

    def _scatter_desc(j):
      return pltpu.make_async_copy(rows.at[j % NB], h_shared.at[didx.at[j]],
                                   ssem.at[j % NB])

    def _chunk_ops(i, j, first, last):
      nx_j = (j + 2) % 6
      if not last:
        if not first or j + 2 >= NB:
          _scatter_desc((j + 5) % 6).wait()
        for dsc in _idx_descs(i + 2, nx_j):
          dsc.wait()
        _gather_desc(nx_j).start()
      _gather_desc(j).wait()
      _scatter_desc(j).start(add=True)
      if not last:
        for dsc in _idx_descs(i + 5, (j + 5) % 6):
          dsc.start()

    for j in range(5):
      for dsc in _idx_descs(j, j):
        dsc.start()
    for j in range(2):
      for dsc in _idx_descs(j, j):
        dsc.wait()
      _gather_desc(j).start()

    for i in range(6):
      _chunk_ops(i, i, first=(i < 1), last=False)

    def _agg_group(it, _):
      i0 = it * 6
      for j in range(6):
        _chunk_ops(i0 + j, j, first=False, last=False)
      return _
    lax.fori_loop(1, 20, _agg_group, None)

    for i in range(120, NCHUNK):
      j = i % 6
      if i + 2 < NCHUNK:
        _scatter_desc((j + 5) % 6).wait()
        for dsc in _idx_descs(i + 2, (j + 2) % 6):
          dsc.wait()
        _gather_desc((j + 2) % 6).start()
      _gather_desc(j).wait()
      _scatter_desc(j).start(add=True)
    for i in range(NCHUNK - NB, NCHUNK):
      _scatter_desc(i % 6).wait()
    plsc.subcore_barrier()

    def _eblk(r0, nb):
      _load_d(r0, nb)
      pltpu.sync_copy(h_shared.at[pl.ds(r0, nb), :], tbuf.at[pl.ds(0, nb), :])
      _scale_rows(nb)
      pltpu.sync_copy(tbuf.at[pl.ds(0, nb), :],
                      hp_hbm.at[c, pl.ds(r0, nb), :])

    _over_blocks(_eblk)

  return fused


def _update_body(hp_ref, x0_ref, w1_ref, out_ref):
  m = (1.0 - ALPHA) * (hp_ref[0] + hp_ref[1]) + ALPHA * x0_ref[...]
  row = lax.broadcasted_iota(jnp.int32, (C, C), 0)
  col = lax.broadcasted_iota(jnp.int32, (C, C), 1)
  eye = jnp.where(row == col, 1.0, 0.0).astype(jnp.float32)
  wmod = (1.0 - BETA) * eye + BETA * w1_ref[...]
  out_ref[...] = jnp.dot(m, wmod, preferred_element_type=jnp.float32)


def kernel(x, x_0, edge_index, weight1):
  src = edge_index[0].astype(jnp.int32)
  dst = edge_index[1].astype(jnp.int32)

  hp, _ = _make_fused_kernel()(x, src, dst)

  rblk = 1000
  out = pl.pallas_call(
      _update_body,
      grid=(N // rblk,),
      in_specs=[
          pl.BlockSpec((NC, rblk, C), lambda i: (0, i, 0)),
          pl.BlockSpec((rblk, C), lambda i: (i, 0)),
          pl.BlockSpec((C, C), lambda i: (0, 0)),
      ],
      out_specs=pl.BlockSpec((rblk, C), lambda i: (i, 0)),
      out_shape=jax.ShapeDtypeStruct((N, C), jnp.float32),
  )(hp, x_0, weight1)
  return out


# speedup vs baseline: 1.0679x; 1.0679x over previous
import functools
import math

import jax
import jax.numpy as jnp
from jax import lax
from jax.experimental import pallas as pl
from jax.experimental.pallas import tpu as pltpu
from jax.experimental.pallas import tpu_sc as plsc

N = 10000
E = 320000
C = 128
ALPHA = 0.1
BETA = float(math.log(0.5 / 1 + 1))

NC = 2
NS = 16
NW = NC * NS

K = 80
EPT = E // NW
NCHUNK = EPT // K
EPC = E // NS
NCHUNK_D = EPC // K
NB = 3
NI = 6

RPT = 624
RLAST = 640
EBLK = 104
EBLK_L = 80


def _rsqrt_newton(v):
  s = v
  for _ in range(16):
    s = 0.5 * (s + v / s)
  return 1.0 / s


def _make_deg_scale_kernel():
  mesh = plsc.VectorSubcoreMesh(core_axis_name="c", subcore_axis_name="s")

  @functools.partial(
      pl.kernel,
      mesh=mesh,
      out_type=(
          jax.ShapeDtypeStruct((N,), jnp.float32),
          jax.ShapeDtypeStruct((N, C), jnp.float32),
      ),
      scratch_types=[
          pltpu.VMEM_SHARED((N,), jnp.float32),
          pltpu.VMEM((20, K), jnp.int32),
          pltpu.VMEM((K,), jnp.float32),
          pltpu.VMEM((640,), jnp.float32),
          pltpu.VMEM((640,), jnp.float32),
          pltpu.VMEM((640,), jnp.float32),
          pltpu.VMEM((320, C), jnp.float32),
          pltpu.SemaphoreType.DMA((21,)),
      ],
      compiler_params=pltpu.CompilerParams(needs_layout_passes=False),
  )
  def deg_scale(dst_hbm, x_hbm, d_hbm, xs_hbm,
                cnt_shared, dring, ones, zsrc, degloc, dloc, xbuf, dsem):
    c = lax.axis_index("c")
    s = lax.axis_index("s")
    w = c * NS + s

    def _fill(i, _):
      zsrc[pl.ds(i * 16, 16)] = jnp.zeros((16,), jnp.float32)
      return _
    lax.fori_loop(0, 40, _fill, None)
    for j in range(K // 16):
      ones[pl.ds(j * 16, 16)] = jnp.ones((16,), jnp.float32)

    @pl.when(s < NS - 1)
    def _():
      pltpu.sync_copy(zsrc.at[pl.ds(0, 640)], cnt_shared.at[pl.ds(s * 640, 640)])

    @pl.when(s == NS - 1)
    def _():
      pltpu.sync_copy(zsrc.at[pl.ds(0, 400)], cnt_shared.at[pl.ds(9600, 400)])

    plsc.subcore_barrier()

    GSZ = 10
    NGRP = NCHUNK_D // GSZ

    def _deg_load_desc(i, sl):
      return pltpu.make_async_copy(
          dst_hbm.at[pl.ds(s * EPC + i * K, K)], dring.at[sl], dsem.at[sl])

    def _deg_scat_desc(sl):
      return pltpu.make_async_copy(ones, cnt_shared.at[dring.at[sl]],
                                   dsem.at[2 * GSZ])

    def _deg_halfgroup(g, base):
      for j in range(GSZ):
        _deg_load_desc(g * GSZ + j, base + j).wait()
      for j in range(GSZ):
        _deg_scat_desc(base + j).start(add=True)
      for j in range(GSZ):
        _deg_scat_desc(base + j).wait()

    for j in range(GSZ):
      _deg_load_desc(j, j).start()

    def _deg_group(g2, _):
      ga = 2 * g2
      for j in range(GSZ):
        _deg_load_desc((ga + 1) * GSZ + j, GSZ + j).start()
      _deg_halfgroup(ga, 0)
      for j in range(GSZ):
        _deg_load_desc((ga + 2) * GSZ + j, j).start()
      _deg_halfgroup(ga + 1, GSZ)
      return _

    lax.fori_loop(0, (NGRP - 1) // 2, _deg_group, None)
    _deg_halfgroup(NGRP - 1, 0)
    plsc.subcore_barrier()

    def _epi(r0, nr):
      pltpu.sync_copy(cnt_shared.at[pl.ds(r0, nr)], degloc.at[pl.ds(0, nr)])

      def _dchunk(k, _):
        deg = degloc[pl.ds(k * 16, 16)] + 1.0
        dloc[pl.ds(k * 16, 16)] = _rsqrt_newton(deg)
        return _
      lax.fori_loop(0, nr // 16, _dchunk, None)

      pltpu.sync_copy(x_hbm.at[pl.ds(r0, nr), :], xbuf.at[pl.ds(0, nr), :])

      def _row(r, _):
        idx = jnp.broadcast_to(r, (16,)).astype(jnp.int32)
        dv = plsc.load_gather(dloc, [idx])
        for j in range(C // 16):
          sl = pl.ds(j * 16, 16)
          xbuf[r, sl] = xbuf[r, sl] * dv
        return _
      lax.fori_loop(0, nr, _row, None)

      pltpu.sync_copy(xbuf.at[pl.ds(0, nr), :], xs_hbm.at[pl.ds(r0, nr), :])
      pltpu.sync_copy(dloc.at[pl.ds(0, nr)], d_hbm.at[pl.ds(r0, nr)])

    @pl.when(w < NW - 1)
    def _():
      _epi(w * 320, 320)

    @pl.when(w == NW - 1)
    def _():
      _epi((NW - 1) * 320, 80)

  return deg_scale


def _make_agg_kernel():
  mesh = plsc.VectorSubcoreMesh(core_axis_name="c", subcore_axis_name="s")

  @functools.partial(
      pl.kernel,
      mesh=mesh,
      out_type=jax.ShapeDtypeStruct((NC, N, C), jnp.float32),
      scratch_types=[
          pltpu.VMEM_SHARED((N, C), jnp.float32),
          pltpu.VMEM((NI, K), jnp.int32),
          pltpu.VMEM((NI, K), jnp.int32),
          pltpu.VMEM((NB, K, C), jnp.float32),
          pltpu.VMEM((EBLK, C), jnp.float32),
          pltpu.VMEM((EBLK,), jnp.float32),
          pltpu.SemaphoreType.DMA((NI,)),
          pltpu.SemaphoreType.DMA((NB,)),
          pltpu.SemaphoreType.DMA((NB,)),
      ],
      compiler_params=pltpu.CompilerParams(needs_layout_passes=False),
  )
  def agg(xs_hbm, src_hbm, dst_hbm, d_hbm, hp_hbm,
          h_shared, sidx, didx, rows, tbuf, dbuf, isem, gsem, ssem):
    c = lax.axis_index("c")
    s = lax.axis_index("s")
    w = c * NS + s

    def _zero_tbuf(r, _):
      for j in range(C // 16):
        tbuf[r, pl.ds(j * 16, 16)] = jnp.zeros((16,), jnp.float32)
      return _

    def _init(r0, nrow, blk):
      @pl.when(c == 0)
      def _():
        pltpu.sync_copy(xs_hbm.at[pl.ds(r0, nrow), :],
                        h_shared.at[pl.ds(r0, nrow), :])

      @pl.when(c == 1)
      def _():
        for b in range(nrow // blk):
          pltpu.sync_copy(tbuf.at[pl.ds(0, blk), :],
                          h_shared.at[pl.ds(r0 + b * blk, blk), :])

    lax.fori_loop(0, EBLK, _zero_tbuf, None)

    @pl.when(s < NS - 1)
    def _():
      _init(s * RPT, RPT, EBLK)

    @pl.when(s == NS - 1)
    def _():
      _init((NS - 1) * RPT, RLAST, EBLK_L)

    plsc.subcore_barrier()

    def _idx_start(i):
      j = i % NI
      d1 = pltpu.make_async_copy(src_hbm.at[pl.ds(w * EPT + i * K, K)],
                                 sidx.at[j], isem.at[j])
      d2 = pltpu.make_async_copy(dst_hbm.at[pl.ds(w * EPT + i * K, K)],
                                 didx.at[j], isem.at[j])
      d1.start()
      d2.start()
      return (d1, d2)

    def _gather_start(i):
      b = i % NB
      dsc = pltpu.make_async_copy(xs_hbm.at[sidx.at[i % NI]], rows.at[b],
                                  gsem.at[b])
      dsc.start()
      return dsc

    def _scatter_start(i):
      b = i % NB
      dsc = pltpu.make_async_copy(rows.at[b], h_shared.at[didx.at[i % NI]],
                                  ssem.at[b])
      dsc.start(add=True)
      return dsc

    def _idx_descs(i, j):
      d1 = pltpu.make_async_copy(src_hbm.at[pl.ds(w * EPT + i * K, K)],
                                 sidx.at[j], isem.at[j])
      d2 = pltpu.make_async_copy(dst_hbm.at[pl.ds(w * EPT + i * K, K)],
                                 didx.at[j], isem.at[j])
      return (d1, d2)

    def _gather_desc(j):
      return pltpu.make_async_copy(xs_hbm.at[sidx.at[j]], rows.at[j % NB],
                                   gsem.at[j % NB])

    def _scatter_desc(j):
      return pltpu.make_async_copy(rows.at[j % NB], h_shared.at[didx.at[j]],
                                   ssem.at[j % NB])

    def _chunk_ops(i, j, first, last):
      nx_j = (j + 2) % 6
      if not last:
        if not first or j + 2 >= NB:
          _scatter_desc((j + 5) % 6).wait()
        for dsc in _idx_descs(i + 2, nx_j):
          dsc.wait()
        _gather_desc(nx_j).start()
      _gather_desc(j).wait()
      _scatter_desc(j).start(add=True)
      if not last:
        for dsc in _idx_descs(i + 5, (j + 5) % 6):
          dsc.start()

    for j in range(5):
      for dsc in _idx_descs(j, j):
        dsc.start()
    for j in range(2):
      for dsc in _idx_descs(j, j):
        dsc.wait()
      _gather_desc(j).start()

    for i in range(6):
      _chunk_ops(i, i, first=(i < 1), last=False)

    def _agg_group(it, _):
      i0 = it * 6
      for j in range(6):
        _chunk_ops(i0 + j, j, first=False, last=False)
      return _
    lax.fori_loop(1, 20, _agg_group, None)

    for i in range(120, NCHUNK):
      j = i % 6
      if i + 2 < NCHUNK:
        _scatter_desc((j + 5) % 6).wait()
        for dsc in _idx_descs(i + 2, (j + 2) % 6):
          dsc.wait()
        _gather_desc((j + 2) % 6).start()
      _gather_desc(j).wait()
      _scatter_desc(j).start(add=True)
    for i in range(NCHUNK - NB, NCHUNK):
      _scatter_desc(i % 6).wait()
    plsc.subcore_barrier()

    def _eblk(r0, nb):
      pltpu.sync_copy(h_shared.at[pl.ds(r0, nb), :], tbuf.at[pl.ds(0, nb), :])
      pltpu.sync_copy(d_hbm.at[pl.ds(r0, nb)], dbuf.at[pl.ds(0, nb)])

      def _row(r, _):
        idx = jnp.broadcast_to(r, (16,)).astype(jnp.int32)
        dv = plsc.load_gather(dbuf, [idx])
        for j in range(C // 16):
          sl = pl.ds(j * 16, 16)
          tbuf[r, sl] = tbuf[r, sl] * dv
        return _
      lax.fori_loop(0, nb, _row, None)

      pltpu.sync_copy(tbuf.at[pl.ds(0, nb), :],
                      hp_hbm.at[c, pl.ds(r0, nb), :])

    @pl.when(s < NS - 1)
    def _():
      def _b(bk, _):
        _eblk(s * RPT + bk * EBLK, EBLK)
        return _
      lax.fori_loop(0, RPT // EBLK, _b, None)

    @pl.when(s == NS - 1)
    def _():
      def _b(bk, _):
        _eblk((NS - 1) * RPT + bk * EBLK_L, EBLK_L)
        return _
      lax.fori_loop(0, RLAST // EBLK_L, _b, None)

  return agg


def _update_body(hp_ref, x0_ref, w1_ref, out_ref):
  m = (1.0 - ALPHA) * (hp_ref[0] + hp_ref[1]) + ALPHA * x0_ref[...]
  row = lax.broadcasted_iota(jnp.int32, (C, C), 0)
  col = lax.broadcasted_iota(jnp.int32, (C, C), 1)
  eye = jnp.where(row == col, 1.0, 0.0).astype(jnp.float32)
  wmod = (1.0 - BETA) * eye + BETA * w1_ref[...]
  out_ref[...] = jnp.dot(m, wmod, preferred_element_type=jnp.float32)


def kernel(x, x_0, edge_index, weight1):
  src = edge_index[0].astype(jnp.int32)
  dst = edge_index[1].astype(jnp.int32)

  d, xs = _make_deg_scale_kernel()(dst, x)
  hp = _make_agg_kernel()(xs, src, dst, d)

  rblk = 1000
  out = pl.pallas_call(
      _update_body,
      grid=(N // rblk,),
      in_specs=[
          pl.BlockSpec((NC, rblk, C), lambda i: (0, i, 0)),
          pl.BlockSpec((rblk, C), lambda i: (i, 0)),
          pl.BlockSpec((C, C), lambda i: (0, 0)),
      ],
      out_specs=pl.BlockSpec((rblk, C), lambda i: (i, 0)),
      out_shape=jax.ShapeDtypeStruct((N, C), jnp.float32),
  )(hp, x_0, weight1)
  return out
